# baseline (speedup 1.0000x reference)
import functools

import jax
import jax.numpy as jnp
from jax import lax
from jax.experimental import pallas as pl
from jax.experimental.pallas import tpu as pltpu
from jax.experimental.pallas import tpu_sc as plsc

N = 10000
E = 320000
D = 128
CKV = 64
H = 4
HD = CKV // H
L = 16
NC, NS = 2, 16
NW = NC * NS
EPT = E // NW
C = 80
NCHUNK = EPT // C
NP = 10240
RPT = NP // NS
ZR = 128
AW = 128


def _proj_body(x_ref, wkv_ref, wq_ref, kv_ref, q_ref):
    x = x_ref[...]
    kv_ref[...] = jnp.dot(x, wkv_ref[...], preferred_element_type=jnp.float32)
    wq = jnp.concatenate([wq_ref[...], jnp.zeros((D, D - CKV), jnp.float32)], axis=1)
    q_ref[...] = jnp.dot(x, wq, preferred_element_type=jnp.float32)


def _edge_body(kv_hbm, q_hbm, src_hbm, dst_hbm, out_hbm,
               src_v, dst_v, kvb, qb, contrib, zbuf, acc, sem1, sem2):
    cid = lax.axis_index("c")
    sid = lax.axis_index("s")
    wid = cid * NS + sid

    @pl.loop(0, ZR)
    def _zero(r):
        for j in range(AW // L):
            zbuf[r, pl.ds(j * L, L)] = jnp.zeros((L,), jnp.float32)

    for k in range(RPT // ZR):
        pltpu.sync_copy(zbuf, acc.at[pl.ds(sid * RPT + k * ZR, ZR)])
    plsc.subcore_barrier()

    base = wid * EPT

    @pl.loop(0, NCHUNK)
    def _chunk(i):
        off = base + i * C
        c1 = pltpu.async_copy(src_hbm.at[pl.ds(off, C)], src_v, sem1)
        c2 = pltpu.async_copy(dst_hbm.at[pl.ds(off, C)], dst_v, sem2)
        c1.wait()
        c2.wait()
        g1 = pltpu.async_copy(kv_hbm.at[src_v], kvb, sem1)
        g2 = pltpu.async_copy(q_hbm.at[dst_v], qb, sem2)
        g1.wait()
        g2.wait()

        @pl.loop(0, C, unroll=2)
        def _edge(e):
            lane = lax.iota(jnp.int32, L)
            ewvec = jnp.zeros((L,), jnp.float32)
            for h in range(H):
                kh = kvb[e, pl.ds(CKV + h * HD, HD)]
                qh = qb[e, pl.ds(h * HD, HD)]
                s = jnp.sum(kh * qh) * 0.125
                ew = jnp.exp(jnp.full((L,), s, jnp.float32))
                vh = kvb[e, pl.ds(h * HD, HD)]
                contrib[e, pl.ds(h * HD, HD)] = ew * vh
                ewvec = jnp.where(lane == h, ew, ewvec)
            contrib[e, pl.ds(CKV, L)] = ewvec

        pltpu.sync_copy(contrib, acc.at[dst_v], add=True)

    plsc.subcore_barrier()
    pltpu.sync_copy(acc.at[pl.ds(sid * RPT, RPT)],
                    out_hbm.at[cid, pl.ds(sid * RPT, RPT)])


def _combine_body(x_ref, acc_ref, wp_ref, o_ref):
    a = acc_ref[0, :N] + acc_ref[1, :N]
    znum = a[:, :CKV]
    ssum = a[:, CKV:CKV + H]
    ssum = jnp.where(ssum == 0.0, 1.0, ssum)
    rh = lax.broadcasted_iota(jnp.int32, (H, CKV), 0)
    rc = lax.broadcasted_iota(jnp.int32, (H, CKV), 1) // HD
    expand = jnp.where(rh == rc, 1.0, 0.0)
    denom = jnp.dot(ssum, expand, preferred_element_type=jnp.float32)
    z = znum / denom
    o_ref[...] = (jnp.dot(x_ref[...], wp_ref[:D], preferred_element_type=jnp.float32)
                  + jnp.dot(z, wp_ref[D:], preferred_element_type=jnp.float32))


@functools.cache
def _edge_kernel():
    mesh = plsc.VectorSubcoreMesh(
        core_axis_name="c", subcore_axis_name="s",
        num_cores=NC, num_subcores=NS)
    return pl.kernel(
        _edge_body,
        out_type=jax.ShapeDtypeStruct((NC, NP, AW), jnp.float32),
        mesh=mesh,
        compiler_params=pltpu.CompilerParams(needs_layout_passes=False),
        scratch_types=[
            pltpu.VMEM((C,), jnp.int32),
            pltpu.VMEM((C,), jnp.int32),
            pltpu.VMEM((C, D), jnp.float32),
            pltpu.VMEM((C, D), jnp.float32),
            pltpu.VMEM((C, AW), jnp.float32),
            pltpu.VMEM((ZR, AW), jnp.float32),
            pltpu.VMEM_SHARED((NP, AW), jnp.float32),
            pltpu.SemaphoreType.DMA,
            pltpu.SemaphoreType.DMA,
        ],
    )

_proj_call = pl.pallas_call(
    _proj_body,
    out_shape=[jax.ShapeDtypeStruct((N, D), jnp.float32),
               jax.ShapeDtypeStruct((N, D), jnp.float32)],
)

_combine_call = pl.pallas_call(
    _combine_body,
    out_shape=jax.ShapeDtypeStruct((N, D), jnp.float32),
)


@jax.jit
def _impl(x, edge_index, W_kv, W_q, W_proj):
    src = edge_index[0].astype(jnp.int32)
    dst = edge_index[1].astype(jnp.int32)
    kv_t, q_t = _proj_call(x, W_kv, W_q)
    acc = _edge_kernel()(kv_t, q_t, src, dst)
    return _combine_call(x, acc, W_proj)


def kernel(x, edge_index, W_kv, W_q, W_proj):
    return _impl(x, edge_index, W_kv, W_q, W_proj)


# speedup vs baseline: 3.7340x; 3.7340x over previous
import functools

import jax
import jax.numpy as jnp
from jax import lax
from jax.experimental import pallas as pl
from jax.experimental.pallas import tpu as pltpu
from jax.experimental.pallas import tpu_sc as plsc

N = 10000
E = 320000
D = 128
CKV = 64
H = 4
HD = CKV // H
L = 16
NC, NS = 2, 16
NW = NC * NS
EPT = E // NW
C = 80
NCHUNK = EPT // C
NP = 10240
RPT = NP // NS
ZR = 128
AW = 128


def _proj_body(x_ref, wkv_ref, wq_ref, kv_ref, q_ref):
    x = x_ref[...]
    kv_ref[...] = jnp.dot(x, wkv_ref[...], preferred_element_type=jnp.float32)
    wq = jnp.concatenate([wq_ref[...], jnp.zeros((D, D - CKV), jnp.float32)], axis=1)
    q_ref[...] = jnp.dot(x, wq, preferred_element_type=jnp.float32)


def _edge_body(kv_hbm, q_hbm, src_hbm, dst_hbm, out_hbm,
               src_v, dst_v, kvb, qb, contrib, zbuf, acc, sem1, sem2):
    cid = lax.axis_index("c")
    sid = lax.axis_index("s")
    wid = cid * NS + sid

    @pl.loop(0, ZR)
    def _zero(r):
        for j in range(AW // L):
            zbuf[r, pl.ds(j * L, L)] = jnp.zeros((L,), jnp.float32)

    for k in range(RPT // ZR):
        pltpu.sync_copy(zbuf, acc.at[pl.ds(sid * RPT + k * ZR, ZR)])
    plsc.subcore_barrier()

    base = wid * EPT

    @pl.loop(0, NCHUNK)
    def _chunk(i):
        off = base + i * C
        c1 = pltpu.async_copy(src_hbm.at[pl.ds(off, C)], src_v, sem1)
        c2 = pltpu.async_copy(dst_hbm.at[pl.ds(off, C)], dst_v, sem2)
        c1.wait()
        c2.wait()
        g1 = pltpu.async_copy(kv_hbm.at[src_v], kvb, sem1)
        g2 = pltpu.async_copy(q_hbm.at[dst_v], qb, sem2)
        g1.wait()
        g2.wait()

        @pl.loop(0, C)
        def _edge(e):
            lane = lax.iota(jnp.int32, L)
            ewvec = jnp.zeros((L,), jnp.float32)
            for h in range(H):
                kh = kvb[e, pl.ds(CKV + h * HD, HD)]
                qh = qb[e, pl.ds(h * HD, HD)]
                s = jnp.sum(kh * qh) * 0.125
                ew = jnp.exp(jnp.full((L,), s, jnp.float32))
                vh = kvb[e, pl.ds(h * HD, HD)]
                contrib[e, pl.ds(h * HD, HD)] = ew * vh
                ewvec = jnp.where(lane == h, ew, ewvec)
            contrib[e, pl.ds(CKV, L)] = ewvec

        pltpu.sync_copy(contrib, acc.at[dst_v], add=True)

    plsc.subcore_barrier()
    pltpu.sync_copy(acc.at[pl.ds(sid * RPT, RPT)],
                    out_hbm.at[cid, pl.ds(sid * RPT, RPT)])


def _combine_body(x_ref, acc_ref, wp_ref, o_ref):
    a = acc_ref[0, :N] + acc_ref[1, :N]
    znum = a[:, :CKV]
    ssum = a[:, CKV:CKV + H]
    ssum = jnp.where(ssum == 0.0, 1.0, ssum)
    rh = lax.broadcasted_iota(jnp.int32, (H, CKV), 0)
    rc = lax.broadcasted_iota(jnp.int32, (H, CKV), 1) // HD
    expand = jnp.where(rh == rc, 1.0, 0.0)
    denom = jnp.dot(ssum, expand, preferred_element_type=jnp.float32)
    z = znum / denom
    o_ref[...] = (jnp.dot(x_ref[...], wp_ref[:D], preferred_element_type=jnp.float32)
                  + jnp.dot(z, wp_ref[D:], preferred_element_type=jnp.float32))


@functools.cache
def _edge_kernel():
    mesh = plsc.VectorSubcoreMesh(
        core_axis_name="c", subcore_axis_name="s",
        num_cores=NC, num_subcores=NS)
    return pl.kernel(
        _edge_body,
        out_type=jax.ShapeDtypeStruct((NC, NP, AW), jnp.float32),
        mesh=mesh,
        compiler_params=pltpu.CompilerParams(needs_layout_passes=False),
        scratch_types=[
            pltpu.VMEM((C,), jnp.int32),
            pltpu.VMEM((C,), jnp.int32),
            pltpu.VMEM((C, D), jnp.float32),
            pltpu.VMEM((C, D), jnp.float32),
            pltpu.VMEM((C, AW), jnp.float32),
            pltpu.VMEM((ZR, AW), jnp.float32),
            pltpu.VMEM_SHARED((NP, AW), jnp.float32),
            pltpu.SemaphoreType.DMA,
            pltpu.SemaphoreType.DMA,
        ],
    )

_proj_call = pl.pallas_call(
    _proj_body,
    out_shape=[jax.ShapeDtypeStruct((N, D), jnp.float32),
               jax.ShapeDtypeStruct((N, D), jnp.float32)],
)

_combine_call = pl.pallas_call(
    _combine_body,
    out_shape=jax.ShapeDtypeStruct((N, D), jnp.float32),
)


@jax.jit
def _impl(x, edge_index, W_kv, W_q, W_proj):
    src = edge_index[0].astype(jnp.int32)
    dst = edge_index[1].astype(jnp.int32)
    kv_t, q_t = _proj_call(x, W_kv, W_q)
    acc = _edge_kernel()(kv_t, q_t, src, dst)
    return _combine_call(x, acc, W_proj)


def kernel(x, edge_index, W_kv, W_q, W_proj):
    return _impl(x, edge_index, W_kv, W_q, W_proj)
